# Initial kernel scaffold; baseline (speedup 1.0000x reference)
import jax
import jax.numpy as jnp
from jax.experimental import pallas as pl


def kernel(indices, emb, W, b):
    raise NotImplementedError("write your pallas kernel here")


# speedup vs baseline: 19.0732x; 19.0732x over previous
import functools

import jax
import jax.numpy as jnp
from jax import lax
from jax.experimental import pallas as pl
from jax.experimental.pallas import tpu as pltpu
from jax.experimental.pallas import tpu_sc as plsc


def _sc_counts(B, L, V, NW):
    tb = B // NW
    ng = tb // 16
    mesh = plsc.VectorSubcoreMesh(core_axis_name="c", subcore_axis_name="s")

    @functools.partial(
        pl.kernel,
        mesh=mesh,
        out_type=jax.ShapeDtypeStruct((B * V,), jnp.float32),
        scratch_types=[
            pltpu.VMEM((tb * L,), jnp.int32),
            pltpu.VMEM((tb * V,), jnp.float32),
        ],
        compiler_params=pltpu.CompilerParams(needs_layout_passes=False),
    )
    def counts_kernel(idx_hbm, cnt_hbm, idx_v, cnt_v):
        nc = lax.axis_size("c")
        wid = lax.axis_index("s") * nc + lax.axis_index("c")
        base = wid * tb
        pltpu.sync_copy(idx_hbm.at[pl.ds(base * L, tb * L)], idx_v)

        lane = lax.iota(jnp.int32, 16)
        ones = jnp.ones((16,), jnp.float32)
        zeros = jnp.zeros((16,), jnp.float32)

        def group(g, carry):
            for r in range(16):
                cnt_v[pl.ds(g * (16 * V) + r * V, V)] = zeros
            idx_base = g * (16 * L) + lane * L
            row_base = g * (16 * V) + lane * V
            for l in range(L):
                vals = plsc.load_gather(idx_v, [idx_base + l])
                plsc.addupdate_scatter(cnt_v, [row_base + vals], ones)
            return carry

        lax.fori_loop(0, ng, group, 0)
        pltpu.sync_copy(cnt_v, cnt_hbm.at[pl.ds(base * V, tb * V)])

    return counts_kernel


def _tc_project(cnt_ref, emb_ref, w_ref, b_ref, out_ref, *, inv_l):
    c = cnt_ref[...]
    e = jnp.dot(c, emb_ref[...], preferred_element_type=jnp.float32) * inv_l
    e = jnp.where(jnp.isfinite(e), e, 0.0)
    o = jnp.dot(e, w_ref[...], preferred_element_type=jnp.float32) + b_ref[...]
    out_ref[...] = jnp.maximum(o, 0.0)


def kernel(indices, emb, W, b):
    B, L = indices.shape
    V, D = emb.shape
    P = W.shape[1]
    NW = 32
    idx_flat = indices.astype(jnp.int32).reshape(B * L)

    counts = _sc_counts(B, L, V, NW)(idx_flat).reshape(B, V)

    BLK = 512
    out = pl.pallas_call(
        functools.partial(_tc_project, inv_l=1.0 / L),
        grid=(B // BLK,),
        in_specs=[
            pl.BlockSpec((BLK, V), lambda i: (i, 0)),
            pl.BlockSpec((V, D), lambda i: (0, 0)),
            pl.BlockSpec((D, P), lambda i: (0, 0)),
            pl.BlockSpec((1, P), lambda i: (0, 0)),
        ],
        out_specs=pl.BlockSpec((BLK, P), lambda i: (i, 0)),
        out_shape=jax.ShapeDtypeStruct((B, P), jnp.float32),
    )(counts, emb, W, b.reshape(1, P))
    return out


# speedup vs baseline: 26.3281x; 1.3804x over previous
import functools

import jax
import jax.numpy as jnp
from jax import lax
from jax.experimental import pallas as pl
from jax.experimental.pallas import tpu as pltpu
from jax.experimental.pallas import tpu_sc as plsc


def _sc_counts(B, L, V, NW):
    tb = B // NW
    ng = tb // 16
    mesh = plsc.VectorSubcoreMesh(core_axis_name="c", subcore_axis_name="s")

    @functools.partial(
        pl.kernel,
        mesh=mesh,
        out_type=jax.ShapeDtypeStruct((B, V), jnp.float32),
        scratch_types=[
            pltpu.VMEM((tb, L), jnp.int32),
            pltpu.VMEM((tb, V), jnp.float32),
        ],
        compiler_params=pltpu.CompilerParams(needs_layout_passes=False),
    )
    def counts_kernel(idx_hbm, cnt_hbm, idx_v, cnt_v):
        nc = lax.axis_size("c")
        wid = lax.axis_index("s") * nc + lax.axis_index("c")
        base = wid * tb
        pltpu.sync_copy(idx_hbm.at[pl.ds(base, tb)], idx_v)

        lane = lax.iota(jnp.int32, 16)
        ones = jnp.ones((16,), jnp.float32)
        zeros = jnp.zeros((16,), jnp.float32)

        def group(g, carry):
            text = g * 16 + lane
            for r in range(16):
                cnt_v[g * 16 + r, :] = zeros
            for l in range(L):
                col = jnp.full((16,), l, jnp.int32)
                vals = plsc.load_gather(idx_v, [text, col])
                plsc.addupdate_scatter(cnt_v, [text, vals], ones)
            return carry

        lax.fori_loop(0, ng, group, 0)
        pltpu.sync_copy(cnt_v, cnt_hbm.at[pl.ds(base, tb)])

    return counts_kernel


def _tc_project(cnt_ref, emb_ref, w_ref, b_ref, out_ref, p_ref, *, inv_l):
    @pl.when(pl.program_id(0) == 0)
    def _():
        p_ref[...] = jnp.dot(
            emb_ref[...], w_ref[...], preferred_element_type=jnp.float32
        ) * inv_l

    o = jnp.dot(cnt_ref[...], p_ref[...], preferred_element_type=jnp.float32)
    out_ref[...] = jnp.maximum(o + b_ref[...], 0.0)


def kernel(indices, emb, W, b):
    B, L = indices.shape
    V, D = emb.shape
    P = W.shape[1]
    NW = 32
    if indices.dtype != jnp.int32:
        indices = indices.astype(jnp.int32)

    counts = _sc_counts(B, L, V, NW)(indices)

    BLK = 2048
    out = pl.pallas_call(
        functools.partial(_tc_project, inv_l=1.0 / L),
        grid=(B // BLK,),
        in_specs=[
            pl.BlockSpec((BLK, V), lambda i: (i, 0)),
            pl.BlockSpec((V, D), lambda i: (0, 0)),
            pl.BlockSpec((D, P), lambda i: (0, 0)),
            pl.BlockSpec((1, P), lambda i: (0, 0)),
        ],
        out_specs=pl.BlockSpec((BLK, P), lambda i: (i, 0)),
        out_shape=jax.ShapeDtypeStruct((B, P), jnp.float32),
        scratch_shapes=[pltpu.VMEM((V, P), jnp.float32)],
    )(counts, emb, W, b.reshape(1, P))
    return out


# speedup vs baseline: 33.5235x; 1.2733x over previous
import functools

import jax
import jax.numpy as jnp
from jax import lax
from jax.experimental import pallas as pl
from jax.experimental.pallas import tpu as pltpu
from jax.experimental.pallas import tpu_sc as plsc


def _sc_counts(B, L, V, NW):
    tb = B // NW
    ng = tb // 16
    mesh = plsc.VectorSubcoreMesh(core_axis_name="c", subcore_axis_name="s")

    @functools.partial(
        pl.kernel,
        mesh=mesh,
        out_type=jax.ShapeDtypeStruct((B, V), jnp.float32),
        scratch_types=[
            pltpu.VMEM((L, tb), jnp.int32),
            pltpu.VMEM((tb, V), jnp.float32),
        ],
        compiler_params=pltpu.CompilerParams(needs_layout_passes=False),
    )
    def counts_kernel(idx_hbm, cnt_hbm, idx_v, cnt_v):
        nc = lax.axis_size("c")
        wid = lax.axis_index("s") * nc + lax.axis_index("c")
        base = wid * tb
        pltpu.sync_copy(idx_hbm.at[:, pl.ds(base, tb)], idx_v)

        lane = lax.iota(jnp.int32, 16)
        ones = jnp.ones((16,), jnp.float32)
        zeros = jnp.zeros((16,), jnp.float32)

        def group(g, carry):
            text = g * 16 + lane
            for r in range(16):
                cnt_v[g * 16 + r, :] = zeros
            for l in range(L):
                vals = idx_v[l, pl.ds(g * 16, 16)]
                plsc.addupdate_scatter(cnt_v, [text, vals], ones)
            return carry

        lax.fori_loop(0, ng, group, 0)
        pltpu.sync_copy(cnt_v, cnt_hbm.at[pl.ds(base, tb)])

    return counts_kernel


def _tc_project(cnt_ref, emb_ref, w_ref, b_ref, out_ref, p_ref, *, inv_l):
    @pl.when(pl.program_id(0) == 0)
    def _():
        p_ref[...] = jnp.dot(
            emb_ref[...], w_ref[...], preferred_element_type=jnp.float32
        ) * inv_l

    o = jnp.dot(cnt_ref[...], p_ref[...], preferred_element_type=jnp.float32)
    out_ref[...] = jnp.maximum(o + b_ref[...], 0.0)


def kernel(indices, emb, W, b):
    B, L = indices.shape
    V, D = emb.shape
    P = W.shape[1]
    NW = 32
    if indices.dtype != jnp.int32:
        indices = indices.astype(jnp.int32)

    counts = _sc_counts(B, L, V, NW)(indices.T)

    BLK = 2048
    out = pl.pallas_call(
        functools.partial(_tc_project, inv_l=1.0 / L),
        grid=(B // BLK,),
        in_specs=[
            pl.BlockSpec((BLK, V), lambda i: (i, 0)),
            pl.BlockSpec((V, D), lambda i: (0, 0)),
            pl.BlockSpec((D, P), lambda i: (0, 0)),
            pl.BlockSpec((1, P), lambda i: (0, 0)),
        ],
        out_specs=pl.BlockSpec((BLK, P), lambda i: (i, 0)),
        out_shape=jax.ShapeDtypeStruct((B, P), jnp.float32),
        scratch_shapes=[pltpu.VMEM((V, P), jnp.float32)],
    )(counts, emb, W, b.reshape(1, P))
    return out


# speedup vs baseline: 34.1876x; 1.0198x over previous
import functools

import jax
import jax.numpy as jnp
from jax import lax
from jax.experimental import pallas as pl
from jax.experimental.pallas import tpu as pltpu
from jax.experimental.pallas import tpu_sc as plsc


def _sc_counts(B, L, V, NW):
    tb = B // NW
    ng = tb // 16
    mesh = plsc.VectorSubcoreMesh(core_axis_name="c", subcore_axis_name="s")

    @functools.partial(
        pl.kernel,
        mesh=mesh,
        out_type=jax.ShapeDtypeStruct((B, V), jnp.float32),
        scratch_types=[
            pltpu.VMEM((L, tb), jnp.int32),
            pltpu.VMEM((tb, V), jnp.float32),
        ],
        compiler_params=pltpu.CompilerParams(needs_layout_passes=False),
    )
    def counts_kernel(idx_hbm, cnt_hbm, idx_v, cnt_v):
        nc = lax.axis_size("c")
        wid = lax.axis_index("s") * nc + lax.axis_index("c")
        base = wid * tb
        pltpu.sync_copy(idx_hbm.at[:, pl.ds(base, tb)], idx_v)

        lane = lax.iota(jnp.int32, 16)
        ones = jnp.ones((16,), jnp.float32)
        zeros = jnp.zeros((16,), jnp.float32)

        def zero(r, carry):
            cnt_v[r, :] = zeros
            return carry

        lax.fori_loop(0, tb, zero, 0)

        def position(l, carry):
            for g in range(ng):
                vals = idx_v[l, pl.ds(g * 16, 16)]
                plsc.addupdate_scatter(cnt_v, [g * 16 + lane, vals], ones)
            return carry

        lax.fori_loop(0, L, position, 0)
        pltpu.sync_copy(cnt_v, cnt_hbm.at[pl.ds(base, tb)])

    return counts_kernel


def _tc_project(cnt_ref, emb_ref, w_ref, b_ref, out_ref, p_ref, *, inv_l):
    @pl.when(pl.program_id(0) == 0)
    def _():
        p_ref[...] = jnp.dot(
            emb_ref[...], w_ref[...], preferred_element_type=jnp.float32
        ) * inv_l

    o = jnp.dot(cnt_ref[...], p_ref[...], preferred_element_type=jnp.float32)
    out_ref[...] = jnp.maximum(o + b_ref[...], 0.0)


def kernel(indices, emb, W, b):
    B, L = indices.shape
    V, D = emb.shape
    P = W.shape[1]
    NW = 32
    if indices.dtype != jnp.int32:
        indices = indices.astype(jnp.int32)

    counts = _sc_counts(B, L, V, NW)(indices.T)

    BLK = 4096
    out = pl.pallas_call(
        functools.partial(_tc_project, inv_l=1.0 / L),
        grid=(B // BLK,),
        in_specs=[
            pl.BlockSpec((BLK, V), lambda i: (i, 0)),
            pl.BlockSpec((V, D), lambda i: (0, 0)),
            pl.BlockSpec((D, P), lambda i: (0, 0)),
            pl.BlockSpec((1, P), lambda i: (0, 0)),
        ],
        out_specs=pl.BlockSpec((BLK, P), lambda i: (i, 0)),
        out_shape=jax.ShapeDtypeStruct((B, P), jnp.float32),
        scratch_shapes=[pltpu.VMEM((V, P), jnp.float32)],
    )(counts, emb, W, b.reshape(1, P))
    return out


# speedup vs baseline: 37.8905x; 1.1083x over previous
import functools

import jax
import jax.numpy as jnp
from jax import lax
from jax.experimental import pallas as pl
from jax.experimental.pallas import tpu as pltpu
from jax.experimental.pallas import tpu_sc as plsc


def _sc_counts(B, L, V, NW):
    tb = B // NW
    ng = tb // 16
    mesh = plsc.VectorSubcoreMesh(core_axis_name="c", subcore_axis_name="s")

    @functools.partial(
        pl.kernel,
        mesh=mesh,
        out_type=jax.ShapeDtypeStruct((B, V), jnp.float32),
        scratch_types=[
            pltpu.VMEM((L, tb), jnp.int32),
            pltpu.VMEM((tb, V), jnp.float32),
        ],
        compiler_params=pltpu.CompilerParams(needs_layout_passes=False),
    )
    def counts_kernel(idx_hbm, cnt_hbm, idx_v, cnt_v):
        nc = lax.axis_size("c")
        wid = lax.axis_index("s") * nc + lax.axis_index("c")
        base = wid * tb
        pltpu.sync_copy(idx_hbm.at[:, pl.ds(base, tb)], idx_v)

        lane = lax.iota(jnp.int32, 16)
        ones = jnp.ones((16,), jnp.float32)
        zeros = jnp.zeros((16,), jnp.float32)

        @plsc.parallel_loop(0, ng, unroll=2)
        def group(g):
            text = g * 16 + lane
            for r in range(16):
                cnt_v[g * 16 + r, :] = zeros
            for l in range(L):
                vals = idx_v[l, pl.ds(g * 16, 16)]
                plsc.addupdate_scatter(cnt_v, [text, vals], ones)
        pltpu.sync_copy(cnt_v, cnt_hbm.at[pl.ds(base, tb)])

    return counts_kernel


def _tc_project(cnt_ref, emb_ref, w_ref, b_ref, out_ref, p_ref, *, inv_l):
    @pl.when(pl.program_id(0) == 0)
    def _():
        p_ref[...] = jnp.dot(
            emb_ref[...], w_ref[...], preferred_element_type=jnp.float32
        ) * inv_l

    o = jnp.dot(cnt_ref[...], p_ref[...], preferred_element_type=jnp.float32)
    out_ref[...] = jnp.maximum(o + b_ref[...], 0.0)


def kernel(indices, emb, W, b):
    B, L = indices.shape
    V, D = emb.shape
    P = W.shape[1]
    NW = 32
    if indices.dtype != jnp.int32:
        indices = indices.astype(jnp.int32)

    counts = _sc_counts(B, L, V, NW)(indices.T)

    BLK = 4096
    out = pl.pallas_call(
        functools.partial(_tc_project, inv_l=1.0 / L),
        grid=(B // BLK,),
        in_specs=[
            pl.BlockSpec((BLK, V), lambda i: (i, 0)),
            pl.BlockSpec((V, D), lambda i: (0, 0)),
            pl.BlockSpec((D, P), lambda i: (0, 0)),
            pl.BlockSpec((1, P), lambda i: (0, 0)),
        ],
        out_specs=pl.BlockSpec((BLK, P), lambda i: (i, 0)),
        out_shape=jax.ShapeDtypeStruct((B, P), jnp.float32),
        scratch_shapes=[pltpu.VMEM((V, P), jnp.float32)],
    )(counts, emb, W, b.reshape(1, P))
    return out
